# baseline (speedup 1.0000x reference)
import functools
import jax
import jax.numpy as jnp
from jax import lax
from jax.experimental import pallas as pl
from jax.experimental.pallas import tpu as pltpu
from jax.experimental.pallas import tpu_sc as plsc

_DT = 0.02

_NC, _NS, _L = 2, 16, 16
_NW = _NC * _NS
_B, _S, _D, _E = 16384, 200, 4, 64
_CT = 128
_NCH = _B // (_NW * _CT)
_SEG = 40
_NSEG = _S // _SEG
_NGRP = _CT // _L
_PT_ROW = 40
_OUTR = 16
_HID, _FP2 = 32, 10


def _route_kernel(xr_ref, wrt_ref, sid_ref, sg_ref):
    wrt = wrt_ref[...]
    iota = jax.lax.broadcasted_iota(jnp.int32, (_E, _CT), 0)
    big = jnp.int32(_E)
    neg_inf = jnp.float32(-jnp.inf)

    def body(s, _):
        x4 = xr_ref[0, s]
        logits = wrt[:, 0:1] * x4[0:1]
        for d in range(1, _D):
            logits = logits + wrt[:, d:d + 1] * x4[d:d + 1]
        m1 = jnp.max(logits, axis=0, keepdims=True)
        idx1 = jnp.min(jnp.where(logits == m1, iota, big), axis=0,
                       keepdims=True)
        masked = jnp.where(iota == idx1, neg_inf, logits)
        m2 = jnp.max(masked, axis=0, keepdims=True)
        idx2 = jnp.min(jnp.where(masked == m2, iota, big), axis=0,
                       keepdims=True)
        sid_ref[0, s] = ((idx1 << 6) | idx2)[0]
        sg_ref[0, s] = (1.0 / (1.0 + jnp.exp(m2 - m1)))[0]
        return 0

    lax.fori_loop(0, _SEG, body, 0, unroll=4)


def _route(xrf, wrt):
    nblk = xrf.shape[0]
    return pl.pallas_call(
        _route_kernel,
        grid=(nblk,),
        in_specs=[
            pl.BlockSpec((1, _SEG, _D, _CT), lambda i: (i, 0, 0, 0)),
            pl.BlockSpec((_E, _D), lambda i: (0, 0)),
        ],
        out_specs=[
            pl.BlockSpec((1, _SEG, _CT), lambda i: (i, 0, 0)),
            pl.BlockSpec((1, _SEG, _CT), lambda i: (i, 0, 0)),
        ],
        out_shape=[
            jax.ShapeDtypeStruct((nblk, _SEG, _CT), jnp.int32),
            jax.ShapeDtypeStruct((nblk, _SEG, _CT), jnp.float32),
        ],
        compiler_params=pltpu.CompilerParams(
            dimension_semantics=("parallel",)),
    )(xrf, wrt)


def _tanh(x):
    return 1.0 - 2.0 / (jnp.exp(x + x) + 1.0)


def _gelu(x):
    c0, c1 = 0.7978845608028654, 0.044715
    return 0.5 * x * (1.0 + _tanh(c0 * (x + c1 * x * x * x)))


def _sc_recurrence(xr, sid, sg, ptab, hc):
    mesh = plsc.VectorSubcoreMesh(core_axis_name="c", subcore_axis_name="s")

    @functools.partial(
        pl.kernel, mesh=mesh,
        out_type=jax.ShapeDtypeStruct((_NW, _NCH, _OUTR, _CT), jnp.float32),
        scratch_types=[
            pltpu.VMEM((_SEG, _D, _CT), jnp.float32),
            pltpu.VMEM((_SEG, _CT), jnp.int32),
            pltpu.VMEM((_SEG, _CT), jnp.float32),
            pltpu.VMEM((_E * _PT_ROW,), jnp.float32),
            pltpu.VMEM((512, _L), jnp.float32),
            pltpu.VMEM((_D, _CT), jnp.float32),
            pltpu.VMEM((_OUTR, _CT), jnp.float32),
        ],
        compiler_params=pltpu.CompilerParams(needs_layout_passes=False),
    )
    def body(xr_h, sid_h, sg_h, ptab_h, hc_h, out_h,
             xv, sidv, sgv, ptv, hcv, hv, outv):
        wid = lax.axis_index("s") * _NC + lax.axis_index("c")
        pltpu.sync_copy(ptab_h, ptv)
        pltpu.sync_copy(hc_h, hcv)
        zero16 = jnp.zeros((_L,), jnp.float32)

        def expert_step(base, x, h, g):
            w = [plsc.load_gather(ptv, [base + j]) for j in range(_PT_ROW)]
            hk = []
            for e in range(_D):
                pre = w[32 + e]
                for d in range(_D):
                    pre = pre + x[d] * w[4 * d + e] + h[d] * w[16 + 4 * d + e]
                act = _tanh(pre)
                hk.append(h[e] + w[36 + e] * (act - h[e]))
            return [g * v for v in hk]

        def chunk_body(ch, _):
            def zinit(v, _):
                off = v * _L
                for d in range(_D):
                    hv[d, pl.ds(off, _L)] = zero16
                return 0
            lax.fori_loop(0, _NGRP, zinit, 0)

            def seg_body(seg, _):
                pltpu.sync_copy(xr_h.at[wid, ch, seg], xv)
                pltpu.sync_copy(sid_h.at[wid, ch, seg], sidv)
                pltpu.sync_copy(sg_h.at[wid, ch, seg], sgv)

                def grp_body(v, _):
                    off = v * _L
                    h = [hv[d, pl.ds(off, _L)] for d in range(_D)]

                    def step(t, h):
                        x = [xv[t, d, pl.ds(off, _L)] for d in range(_D)]
                        pid = sidv[t, pl.ds(off, _L)]
                        g1 = sgv[t, pl.ds(off, _L)]
                        i1 = lax.shift_right_logical(pid, 6)
                        i2 = jnp.bitwise_and(pid, 63)
                        hk1 = expert_step(i1 * _PT_ROW, x, list(h), g1)
                        hk2 = expert_step(i2 * _PT_ROW, x, list(h), 1.0 - g1)
                        return tuple(a + b for a, b in zip(hk1, hk2))

                    h = lax.fori_loop(0, _SEG, step, tuple(h))
                    for d in range(_D):
                        hv[d, pl.ds(off, _L)] = h[d]
                    return 0

                lax.fori_loop(0, _NGRP, grp_body, 0)
                return 0

            lax.fori_loop(0, _NSEG, seg_body, 0)

            def head_body(v, _):
                off = v * _L
                h = [hv[d, pl.ds(off, _L)] for d in range(_D)]
                hid = []
                for j in range(_HID):
                    a = hcv[128 + j] + h[0] * hcv[j]
                    for d in range(1, _D):
                        a = a + h[d] * hcv[d * _HID + j]
                    hid.append(_gelu(a))
                for o in range(_FP2):
                    p = hcv[480 + o] + hid[0] * hcv[160 + o]
                    for j in range(1, _HID):
                        p = p + hid[j] * hcv[160 + j * _FP2 + o]
                    outv[o, pl.ds(off, _L)] = 1.0 / (1.0 + jnp.exp(-p))
                return 0

            lax.fori_loop(0, _NGRP, head_body, 0)
            pltpu.sync_copy(outv, out_h.at[wid, ch])
            return 0

        lax.fori_loop(0, _NCH, chunk_body, 0)

    return body(xr, sid, sg, ptab, hc)


def kernel(x_seq, W_router, W_in, W_rec, b, log_tau, head_W1, head_b1,
           head_W2, head_b2):
    xr = x_seq.reshape(_NW, _NCH, _CT, _NSEG, _SEG, _D)
    xr = jnp.transpose(xr, (0, 1, 3, 4, 5, 2))

    sid, sg = _route(xr.reshape(_NW * _NCH * _NSEG, _SEG, _D, _CT),
                     W_router.T)
    sid = sid.reshape(_NW, _NCH, _NSEG, _SEG, _CT)
    sg = sg.reshape(_NW, _NCH, _NSEG, _SEG, _CT)

    ptab = jnp.concatenate(
        [W_in.reshape(_E, 16), W_rec.reshape(_E, 16), b,
         _DT * jnp.exp(-log_tau)], axis=1).reshape(-1)
    hc = jnp.concatenate(
        [head_W1.reshape(-1), head_b1, head_W2.reshape(-1), head_b2,
         jnp.zeros((512 - 490,), jnp.float32)])
    hc = jnp.tile(hc.reshape(512, 1), (1, _L))

    out = _sc_recurrence(xr, sid, sg, ptab, hc)
    y = jnp.transpose(out, (0, 1, 3, 2)).reshape(_B, _OUTR)[:, :_FP2]
    return y.reshape(_B, _FP2 // 2, 2)


# speedup vs baseline: 1.0242x; 1.0242x over previous
import functools
import jax
import jax.numpy as jnp
from jax import lax
from jax.experimental import pallas as pl
from jax.experimental.pallas import tpu as pltpu
from jax.experimental.pallas import tpu_sc as plsc

_DT = 0.02

_NC, _NS, _L = 2, 16, 16
_NW = _NC * _NS
_B, _S, _D, _E = 16384, 200, 4, 64
_CT = 128
_NCH = _B // (_NW * _CT)
_SEG = 40
_NSEG = _S // _SEG
_NGRP = _CT // _L
_PT_ROW = 40
_OUTR = 16
_HID, _FP2 = 32, 10


def _route_kernel(xr_ref, wrt_ref, sid_ref, sg_ref):
    wrt = wrt_ref[...]
    iota = jax.lax.broadcasted_iota(jnp.int32, (_E, _CT), 0)
    big = jnp.int32(_E)
    neg_inf = jnp.float32(-jnp.inf)

    def body(s, _):
        x4 = xr_ref[0, s]
        logits = wrt[:, 0:1] * x4[0:1]
        for d in range(1, _D):
            logits = logits + wrt[:, d:d + 1] * x4[d:d + 1]
        m1 = jnp.max(logits, axis=0, keepdims=True)
        idx1 = jnp.min(jnp.where(logits == m1, iota, big), axis=0,
                       keepdims=True)
        masked = jnp.where(iota == idx1, neg_inf, logits)
        m2 = jnp.max(masked, axis=0, keepdims=True)
        idx2 = jnp.min(jnp.where(masked == m2, iota, big), axis=0,
                       keepdims=True)
        sid_ref[0, s] = ((idx1 << 6) | idx2)[0]
        sg_ref[0, s] = (1.0 / (1.0 + jnp.exp(m2 - m1)))[0]
        return 0

    lax.fori_loop(0, _SEG, body, 0, unroll=4)


def _route(xrf, wrt):
    nblk = xrf.shape[0]
    return pl.pallas_call(
        _route_kernel,
        grid=(nblk,),
        in_specs=[
            pl.BlockSpec((1, _SEG, _D, _CT), lambda i: (i, 0, 0, 0)),
            pl.BlockSpec((_E, _D), lambda i: (0, 0)),
        ],
        out_specs=[
            pl.BlockSpec((1, _SEG, _CT), lambda i: (i, 0, 0)),
            pl.BlockSpec((1, _SEG, _CT), lambda i: (i, 0, 0)),
        ],
        out_shape=[
            jax.ShapeDtypeStruct((nblk, _SEG, _CT), jnp.int32),
            jax.ShapeDtypeStruct((nblk, _SEG, _CT), jnp.float32),
        ],
        compiler_params=pltpu.CompilerParams(
            dimension_semantics=("parallel",)),
    )(xrf, wrt)


def _tanh(x):
    return 1.0 - 2.0 / (jnp.exp(x + x) + 1.0)


def _gelu(x):
    c0, c1 = 0.7978845608028654, 0.044715
    return 0.5 * x * (1.0 + _tanh(c0 * (x + c1 * x * x * x)))


def _sc_recurrence(xr, sid, sg, ptab, hc):
    mesh = plsc.VectorSubcoreMesh(core_axis_name="c", subcore_axis_name="s")

    @functools.partial(
        pl.kernel, mesh=mesh,
        out_type=jax.ShapeDtypeStruct((_NW, _NCH, _OUTR, _CT), jnp.float32),
        scratch_types=[
            pltpu.VMEM((_SEG, _D, _CT), jnp.float32),
            pltpu.VMEM((_SEG, _CT), jnp.int32),
            pltpu.VMEM((_SEG, _CT), jnp.float32),
            pltpu.VMEM((_E * _PT_ROW,), jnp.float32),
            pltpu.VMEM((512, _L), jnp.float32),
            pltpu.VMEM((_D, _CT), jnp.float32),
            pltpu.VMEM((_OUTR, _CT), jnp.float32),
        ],
        compiler_params=pltpu.CompilerParams(needs_layout_passes=False),
    )
    def body(xr_h, sid_h, sg_h, ptab_h, hc_h, out_h,
             xv, sidv, sgv, ptv, hcv, hv, outv):
        wid = lax.axis_index("s") * _NC + lax.axis_index("c")
        pltpu.sync_copy(ptab_h, ptv)
        pltpu.sync_copy(hc_h, hcv)
        zero16 = jnp.zeros((_L,), jnp.float32)

        def expert_step(base, x, h, g):
            w = [plsc.load_gather(ptv, [base + j]) for j in range(_PT_ROW)]
            hk = []
            for e in range(_D):
                pre = w[32 + e]
                for d in range(_D):
                    pre = pre + x[d] * w[4 * d + e] + h[d] * w[16 + 4 * d + e]
                act = _tanh(pre)
                hk.append(h[e] + w[36 + e] * (act - h[e]))
            return [g * v for v in hk]

        def chunk_body(ch, _):
            def zinit(v, _):
                off = v * _L
                for d in range(_D):
                    hv[d, pl.ds(off, _L)] = zero16
                return 0
            lax.fori_loop(0, _NGRP, zinit, 0)

            def seg_body(seg, _):
                pltpu.sync_copy(xr_h.at[wid, ch, seg], xv)
                pltpu.sync_copy(sid_h.at[wid, ch, seg], sidv)
                pltpu.sync_copy(sg_h.at[wid, ch, seg], sgv)

                def grp_body(v, _):
                    off = v * _L
                    h = [hv[d, pl.ds(off, _L)] for d in range(_D)]

                    def step(t, h):
                        x = [xv[t, d, pl.ds(off, _L)] for d in range(_D)]
                        pid = sidv[t, pl.ds(off, _L)]
                        g1 = sgv[t, pl.ds(off, _L)]
                        i1 = lax.shift_right_logical(pid, 6)
                        i2 = jnp.bitwise_and(pid, 63)
                        hk1 = expert_step(i1 * _PT_ROW, x, list(h), g1)
                        hk2 = expert_step(i2 * _PT_ROW, x, list(h), 1.0 - g1)
                        return tuple(a + b for a, b in zip(hk1, hk2))

                    h = lax.fori_loop(0, _SEG, step, tuple(h), unroll=2)
                    for d in range(_D):
                        hv[d, pl.ds(off, _L)] = h[d]
                    return 0

                lax.fori_loop(0, _NGRP, grp_body, 0)
                return 0

            lax.fori_loop(0, _NSEG, seg_body, 0)

            def head_body(v, _):
                off = v * _L
                h = [hv[d, pl.ds(off, _L)] for d in range(_D)]
                hid = []
                for j in range(_HID):
                    a = hcv[128 + j] + h[0] * hcv[j]
                    for d in range(1, _D):
                        a = a + h[d] * hcv[d * _HID + j]
                    hid.append(_gelu(a))
                for o in range(_FP2):
                    p = hcv[480 + o] + hid[0] * hcv[160 + o]
                    for j in range(1, _HID):
                        p = p + hid[j] * hcv[160 + j * _FP2 + o]
                    outv[o, pl.ds(off, _L)] = 1.0 / (1.0 + jnp.exp(-p))
                return 0

            lax.fori_loop(0, _NGRP, head_body, 0)
            pltpu.sync_copy(outv, out_h.at[wid, ch])
            return 0

        lax.fori_loop(0, _NCH, chunk_body, 0)

    return body(xr, sid, sg, ptab, hc)


def kernel(x_seq, W_router, W_in, W_rec, b, log_tau, head_W1, head_b1,
           head_W2, head_b2):
    xr = x_seq.reshape(_NW, _NCH, _CT, _NSEG, _SEG, _D)
    xr = jnp.transpose(xr, (0, 1, 3, 4, 5, 2))

    sid, sg = _route(xr.reshape(_NW * _NCH * _NSEG, _SEG, _D, _CT),
                     W_router.T)
    sid = sid.reshape(_NW, _NCH, _NSEG, _SEG, _CT)
    sg = sg.reshape(_NW, _NCH, _NSEG, _SEG, _CT)

    ptab = jnp.concatenate(
        [W_in.reshape(_E, 16), W_rec.reshape(_E, 16), b,
         _DT * jnp.exp(-log_tau)], axis=1).reshape(-1)
    hc = jnp.concatenate(
        [head_W1.reshape(-1), head_b1, head_W2.reshape(-1), head_b2,
         jnp.zeros((512 - 490,), jnp.float32)])
    hc = jnp.tile(hc.reshape(512, 1), (1, _L))

    out = _sc_recurrence(xr, sid, sg, ptab, hc)
    y = jnp.transpose(out, (0, 1, 3, 2)).reshape(_B, _OUTR)[:, :_FP2]
    return y.reshape(_B, _FP2 // 2, 2)


# speedup vs baseline: 1.7438x; 1.7025x over previous
import functools
import jax
import jax.numpy as jnp
from jax import lax
from jax.experimental import pallas as pl
from jax.experimental.pallas import tpu as pltpu
from jax.experimental.pallas import tpu_sc as plsc

_DT = 0.02

_NC, _NS, _L = 2, 16, 16
_NW = _NC * _NS
_B, _S, _D, _E = 16384, 200, 4, 64
_CT = 128
_B_SC = 4096
_NCH = _B_SC // (_NW * _CT)
_SEG = 40
_NSEG = _S // _SEG
_NGRP = _CT // _L
_PT_ROW = 40
_OUTR = 16
_HID, _FP2 = 32, 10
_B_TILE = 1024


def _route_kernel(xr_ref, wrt_ref, sid_ref, sg_ref):
    wrt = wrt_ref[...]
    iota = jax.lax.broadcasted_iota(jnp.int32, (_E, _CT), 0)
    big = jnp.int32(_E)
    neg_inf = jnp.float32(-jnp.inf)

    def body(s, _):
        x4 = xr_ref[0, s]
        logits = wrt[:, 0:1] * x4[0:1]
        for d in range(1, _D):
            logits = logits + wrt[:, d:d + 1] * x4[d:d + 1]
        m1 = jnp.max(logits, axis=0, keepdims=True)
        idx1 = jnp.min(jnp.where(logits == m1, iota, big), axis=0,
                       keepdims=True)
        masked = jnp.where(iota == idx1, neg_inf, logits)
        m2 = jnp.max(masked, axis=0, keepdims=True)
        idx2 = jnp.min(jnp.where(masked == m2, iota, big), axis=0,
                       keepdims=True)
        sid_ref[0, s] = ((idx1 << 6) | idx2)[0]
        sg_ref[0, s] = (1.0 / (1.0 + jnp.exp(m2 - m1)))[0]
        return 0

    lax.fori_loop(0, _SEG, body, 0, unroll=4)


def _route(xrf, wrt):
    nblk = xrf.shape[0]
    return pl.pallas_call(
        _route_kernel,
        grid=(nblk,),
        in_specs=[
            pl.BlockSpec((1, _SEG, _D, _CT), lambda i: (i, 0, 0, 0)),
            pl.BlockSpec((_E, _D), lambda i: (0, 0)),
        ],
        out_specs=[
            pl.BlockSpec((1, _SEG, _CT), lambda i: (i, 0, 0)),
            pl.BlockSpec((1, _SEG, _CT), lambda i: (i, 0, 0)),
        ],
        out_shape=[
            jax.ShapeDtypeStruct((nblk, _SEG, _CT), jnp.int32),
            jax.ShapeDtypeStruct((nblk, _SEG, _CT), jnp.float32),
        ],
        compiler_params=pltpu.CompilerParams(
            dimension_semantics=("parallel",)),
    )(xrf, wrt)


def _tanh_sc(x):
    return 1.0 - 2.0 / (jnp.exp(x + x) + 1.0)


def _gelu_sc(x):
    c0, c1 = 0.7978845608028654, 0.044715
    return 0.5 * x * (1.0 + _tanh_sc(c0 * (x + c1 * x * x * x)))


def _sc_recurrence(xr, sid, sg, ptab, hc):
    mesh = plsc.VectorSubcoreMesh(core_axis_name="c", subcore_axis_name="s")

    @functools.partial(
        pl.kernel, mesh=mesh,
        out_type=jax.ShapeDtypeStruct((_NW, _NCH, _OUTR, _CT), jnp.float32),
        scratch_types=[
            pltpu.VMEM((_SEG, _D, _CT), jnp.float32),
            pltpu.VMEM((_SEG, _CT), jnp.int32),
            pltpu.VMEM((_SEG, _CT), jnp.float32),
            pltpu.VMEM((_E * _PT_ROW,), jnp.float32),
            pltpu.VMEM((512, _L), jnp.float32),
            pltpu.VMEM((_D, _CT), jnp.float32),
            pltpu.VMEM((_OUTR, _CT), jnp.float32),
        ],
        compiler_params=pltpu.CompilerParams(needs_layout_passes=False),
    )
    def body(xr_h, sid_h, sg_h, ptab_h, hc_h, out_h,
             xv, sidv, sgv, ptv, hcv, hv, outv):
        wid = lax.axis_index("s") * _NC + lax.axis_index("c")
        pltpu.sync_copy(ptab_h, ptv)
        pltpu.sync_copy(hc_h, hcv)
        zero16 = jnp.zeros((_L,), jnp.float32)

        def expert_step(base, x, h, g):
            w = [plsc.load_gather(ptv, [base + j]) for j in range(_PT_ROW)]
            hk = []
            for e in range(_D):
                pre = w[32 + e]
                for d in range(_D):
                    pre = pre + x[d] * w[4 * d + e] + h[d] * w[16 + 4 * d + e]
                act = _tanh_sc(pre)
                hk.append(h[e] + w[36 + e] * (act - h[e]))
            return [g * v for v in hk]

        def chunk_body(ch, _):
            def zinit(v, _):
                off = v * _L
                for d in range(_D):
                    hv[d, pl.ds(off, _L)] = zero16
                return 0
            lax.fori_loop(0, _NGRP, zinit, 0)

            def seg_body(seg, _):
                pltpu.sync_copy(xr_h.at[wid, ch, seg], xv)
                pltpu.sync_copy(sid_h.at[wid, ch, seg], sidv)
                pltpu.sync_copy(sg_h.at[wid, ch, seg], sgv)

                def grp_body(v, _):
                    off = v * _L
                    h = [hv[d, pl.ds(off, _L)] for d in range(_D)]

                    def step(t, h):
                        x = [xv[t, d, pl.ds(off, _L)] for d in range(_D)]
                        pid = sidv[t, pl.ds(off, _L)]
                        g1 = sgv[t, pl.ds(off, _L)]
                        i1 = lax.shift_right_logical(pid, 6)
                        i2 = jnp.bitwise_and(pid, 63)
                        hk1 = expert_step(i1 * _PT_ROW, x, list(h), g1)
                        hk2 = expert_step(i2 * _PT_ROW, x, list(h), 1.0 - g1)
                        return tuple(a + b for a, b in zip(hk1, hk2))

                    h = lax.fori_loop(0, _SEG, step, tuple(h), unroll=2)
                    for d in range(_D):
                        hv[d, pl.ds(off, _L)] = h[d]
                    return 0

                lax.fori_loop(0, _NGRP, grp_body, 0)
                return 0

            lax.fori_loop(0, _NSEG, seg_body, 0)

            def head_body(v, _):
                off = v * _L
                h = [hv[d, pl.ds(off, _L)] for d in range(_D)]
                hid = []
                for j in range(_HID):
                    a = hcv[128 + j] + h[0] * hcv[j]
                    for d in range(1, _D):
                        a = a + h[d] * hcv[d * _HID + j]
                    hid.append(_gelu_sc(a))
                for o in range(_FP2):
                    p = hcv[480 + o] + hid[0] * hcv[160 + o]
                    for j in range(1, _HID):
                        p = p + hid[j] * hcv[160 + j * _FP2 + o]
                    outv[o, pl.ds(off, _L)] = 1.0 / (1.0 + jnp.exp(-p))
                return 0

            lax.fori_loop(0, _NGRP, head_body, 0)
            pltpu.sync_copy(outv, out_h.at[wid, ch])
            return 0

        lax.fori_loop(0, _NCH, chunk_body, 0)

    return body(xr, sid, sg, ptab, hc)


def _liquid_tc_kernel(xT_ref, wrt_ref, p_ref, w1t_ref, b1_ref, w2t_ref,
                      b2_ref, out_ref):
    S = xT_ref.shape[0]
    Bt = xT_ref.shape[2]
    E = wrt_ref.shape[0]

    wrt = wrt_ref[...]
    P = p_ref[...]
    iota = jax.lax.broadcasted_iota(jnp.int32, (E, Bt), 0)
    big = jnp.int32(E)
    neg_inf = jnp.float32(-jnp.inf)

    def expert_apply(oh, x4, h8):
        pk = jnp.dot(P, oh, preferred_element_type=jnp.float32)
        pre = pk[64:72]
        for d in range(4):
            pre = pre + x4[d:d + 1] * pk[8 * d:8 * d + 8] \
                      + h8[d:d + 1] * pk[32 + 8 * d:40 + 8 * d]
        act = jnp.tanh(pre)
        return h8 + pk[72:80] * (act - h8)

    def step(t, h8):
        x4 = xT_ref[t]
        logits = wrt[:, 0:1] * x4[0:1]
        for d in range(1, 4):
            logits = logits + wrt[:, d:d + 1] * x4[d:d + 1]

        m1 = jnp.max(logits, axis=0, keepdims=True)
        idx1 = jnp.min(jnp.where(logits == m1, iota, big), axis=0,
                       keepdims=True)
        oh1 = (iota == idx1)
        masked = jnp.where(oh1, neg_inf, logits)
        m2 = jnp.max(masked, axis=0, keepdims=True)
        idx2 = jnp.min(jnp.where(masked == m2, iota, big), axis=0,
                       keepdims=True)
        oh2 = (iota == idx2)

        e2 = jnp.exp(m2 - m1)
        g1 = 1.0 / (1.0 + e2)
        g2 = e2 * g1

        hk1 = expert_apply(oh1.astype(jnp.float32), x4, h8)
        hk2 = expert_apply(oh2.astype(jnp.float32), x4, h8)
        return g1 * hk1 + g2 * hk2

    h0 = jnp.zeros((8, Bt), dtype=jnp.float32)
    h8 = jax.lax.fori_loop(0, S, step, h0)

    hidden = jnp.dot(w1t_ref[...], h8, preferred_element_type=jnp.float32)
    hidden = jax.nn.gelu(hidden + b1_ref[...])
    pred = jnp.dot(w2t_ref[...], hidden, preferred_element_type=jnp.float32)
    out_ref[...] = jax.nn.sigmoid(pred + b2_ref[...])


def _tc_path(x_tc, W_router, W_in, W_rec, b, log_tau, head_W1, head_b1,
             head_W2, head_b2):
    B, S, D = x_tc.shape
    E = W_router.shape[1]
    HID = head_W1.shape[1]
    FP2 = head_W2.shape[1]

    xT = jnp.transpose(x_tc, (1, 2, 0))

    def pack_dd(W):
        Wt = jnp.transpose(W, (1, 2, 0))
        Wt = jnp.pad(Wt, ((0, 0), (0, 4), (0, 0)))
        return Wt.reshape(8 * D, E)

    b8 = jnp.pad(b.T, ((0, 4), (0, 0)))
    dt8 = jnp.pad(_DT * jnp.exp(-log_tau).T, ((0, 4), (0, 0)))
    P = jnp.concatenate([pack_dd(W_in), pack_dd(W_rec), b8, dt8], axis=0)

    wrt = W_router.T
    w1t = jnp.pad(head_W1.T, ((0, 0), (0, 4)))
    b1c = head_b1.reshape(HID, 1)
    w2t = jnp.pad(head_W2.T, ((0, _OUTR - FP2), (0, 0)))
    b2c = jnp.pad(head_b2, (0, _OUTR - FP2)).reshape(_OUTR, 1)

    bt = min(_B_TILE, B)
    grid = (B // bt,)

    out = pl.pallas_call(
        _liquid_tc_kernel,
        grid=grid,
        in_specs=[
            pl.BlockSpec((S, D, bt), lambda i: (0, 0, i)),
            pl.BlockSpec((E, D), lambda i: (0, 0)),
            pl.BlockSpec((80, E), lambda i: (0, 0)),
            pl.BlockSpec((HID, 8), lambda i: (0, 0)),
            pl.BlockSpec((HID, 1), lambda i: (0, 0)),
            pl.BlockSpec((_OUTR, HID), lambda i: (0, 0)),
            pl.BlockSpec((_OUTR, 1), lambda i: (0, 0)),
        ],
        out_specs=pl.BlockSpec((_OUTR, bt), lambda i: (0, i)),
        out_shape=jax.ShapeDtypeStruct((_OUTR, B), jnp.float32),
        compiler_params=pltpu.CompilerParams(
            dimension_semantics=("parallel",)),
    )(xT, wrt, P, w1t, b1c, w2t, b2c)

    return out[:FP2].T.reshape(B, FP2 // 2, 2)


def kernel(x_seq, W_router, W_in, W_rec, b, log_tau, head_W1, head_b1,
           head_W2, head_b2):
    x_sc = x_seq[:_B_SC]
    x_tc = x_seq[_B_SC:]

    xr = x_sc.reshape(_NW, _NCH, _CT, _NSEG, _SEG, _D)
    xr = jnp.transpose(xr, (0, 1, 3, 4, 5, 2))

    sid, sg = _route(xr.reshape(_NW * _NCH * _NSEG, _SEG, _D, _CT),
                     W_router.T)
    sid = sid.reshape(_NW, _NCH, _NSEG, _SEG, _CT)
    sg = sg.reshape(_NW, _NCH, _NSEG, _SEG, _CT)

    ptab = jnp.concatenate(
        [W_in.reshape(_E, 16), W_rec.reshape(_E, 16), b,
         _DT * jnp.exp(-log_tau)], axis=1).reshape(-1)
    hc = jnp.concatenate(
        [head_W1.reshape(-1), head_b1, head_W2.reshape(-1), head_b2,
         jnp.zeros((512 - 490,), jnp.float32)])
    hc = jnp.tile(hc.reshape(512, 1), (1, _L))

    out_sc = _sc_recurrence(xr, sid, sg, ptab, hc)
    y_sc = jnp.transpose(out_sc, (0, 1, 3, 2)).reshape(_B_SC, _OUTR)
    y_sc = y_sc[:, :_FP2].reshape(_B_SC, _FP2 // 2, 2)

    y_tc = _tc_path(x_tc, W_router, W_in, W_rec, b, log_tau,
                    head_W1, head_b1, head_W2, head_b2)

    return jnp.concatenate([y_sc, y_tc], axis=0)


# speedup vs baseline: 1.7753x; 1.0181x over previous
import functools
import jax
import jax.numpy as jnp
from jax import lax
from jax.experimental import pallas as pl
from jax.experimental.pallas import tpu as pltpu
from jax.experimental.pallas import tpu_sc as plsc

_DT = 0.02

_NC, _NS, _L = 2, 16, 16
_NW = _NC * _NS
_B, _S, _D, _E = 16384, 200, 4, 64
_CT = 128
_B_SC = 8192
_NCH = _B_SC // (_NW * _CT)
_SEG = 40
_NSEG = _S // _SEG
_NGRP = _CT // _L
_PT_ROW = 40
_OUTR = 16
_HID, _FP2 = 32, 10
_B_TILE = 1024


def _route_kernel(xr_ref, wrt_ref, sid_ref, sg_ref):
    wrt = wrt_ref[...]
    iota = jax.lax.broadcasted_iota(jnp.int32, (_E, _CT), 0)
    big = jnp.int32(_E)
    neg_inf = jnp.float32(-jnp.inf)

    def body(s, _):
        x4 = xr_ref[0, s]
        logits = wrt[:, 0:1] * x4[0:1]
        for d in range(1, _D):
            logits = logits + wrt[:, d:d + 1] * x4[d:d + 1]
        m1 = jnp.max(logits, axis=0, keepdims=True)
        idx1 = jnp.min(jnp.where(logits == m1, iota, big), axis=0,
                       keepdims=True)
        masked = jnp.where(iota == idx1, neg_inf, logits)
        m2 = jnp.max(masked, axis=0, keepdims=True)
        idx2 = jnp.min(jnp.where(masked == m2, iota, big), axis=0,
                       keepdims=True)
        sid_ref[0, s] = ((idx1 << 6) | idx2)[0]
        sg_ref[0, s] = (1.0 / (1.0 + jnp.exp(m2 - m1)))[0]
        return 0

    lax.fori_loop(0, _SEG, body, 0, unroll=4)


def _route(xrf, wrt):
    nblk = xrf.shape[0]
    return pl.pallas_call(
        _route_kernel,
        grid=(nblk,),
        in_specs=[
            pl.BlockSpec((1, _SEG, _D, _CT), lambda i: (i, 0, 0, 0)),
            pl.BlockSpec((_E, _D), lambda i: (0, 0)),
        ],
        out_specs=[
            pl.BlockSpec((1, _SEG, _CT), lambda i: (i, 0, 0)),
            pl.BlockSpec((1, _SEG, _CT), lambda i: (i, 0, 0)),
        ],
        out_shape=[
            jax.ShapeDtypeStruct((nblk, _SEG, _CT), jnp.int32),
            jax.ShapeDtypeStruct((nblk, _SEG, _CT), jnp.float32),
        ],
        compiler_params=pltpu.CompilerParams(
            dimension_semantics=("parallel",)),
    )(xrf, wrt)


def _tanh_sc(x):
    return 1.0 - 2.0 / (jnp.exp(x + x) + 1.0)


def _gelu_sc(x):
    c0, c1 = 0.7978845608028654, 0.044715
    return 0.5 * x * (1.0 + _tanh_sc(c0 * (x + c1 * x * x * x)))


def _sc_recurrence(xr, sid, sg, ptab, hc):
    mesh = plsc.VectorSubcoreMesh(core_axis_name="c", subcore_axis_name="s")

    @functools.partial(
        pl.kernel, mesh=mesh,
        out_type=jax.ShapeDtypeStruct((_NW, _NCH, _OUTR, _CT), jnp.float32),
        scratch_types=[
            pltpu.VMEM((_SEG, _D, _CT), jnp.float32),
            pltpu.VMEM((_SEG, _CT), jnp.int32),
            pltpu.VMEM((_SEG, _CT), jnp.float32),
            pltpu.VMEM((_E * _PT_ROW,), jnp.float32),
            pltpu.VMEM((512, _L), jnp.float32),
            pltpu.VMEM((_D, _CT), jnp.float32),
            pltpu.VMEM((_OUTR, _CT), jnp.float32),
        ],
        compiler_params=pltpu.CompilerParams(needs_layout_passes=False),
    )
    def body(xr_h, sid_h, sg_h, ptab_h, hc_h, out_h,
             xv, sidv, sgv, ptv, hcv, hv, outv):
        wid = lax.axis_index("s") * _NC + lax.axis_index("c")
        pltpu.sync_copy(ptab_h, ptv)
        pltpu.sync_copy(hc_h, hcv)
        zero16 = jnp.zeros((_L,), jnp.float32)

        def expert_step(base, x, h, g):
            w = [plsc.load_gather(ptv, [base + j]) for j in range(_PT_ROW)]
            hk = []
            for e in range(_D):
                pre = w[32 + e]
                for d in range(_D):
                    pre = pre + x[d] * w[4 * d + e] + h[d] * w[16 + 4 * d + e]
                act = _tanh_sc(pre)
                hk.append(h[e] + w[36 + e] * (act - h[e]))
            return [g * v for v in hk]

        def chunk_body(ch, _):
            def zinit(v, _):
                off = v * _L
                for d in range(_D):
                    hv[d, pl.ds(off, _L)] = zero16
                return 0
            lax.fori_loop(0, _NGRP, zinit, 0)

            def seg_body(seg, _):
                pltpu.sync_copy(xr_h.at[wid, ch, seg], xv)
                pltpu.sync_copy(sid_h.at[wid, ch, seg], sidv)
                pltpu.sync_copy(sg_h.at[wid, ch, seg], sgv)

                def grp_body(v, _):
                    off = v * _L
                    h = [hv[d, pl.ds(off, _L)] for d in range(_D)]

                    def step(t, h):
                        x = [xv[t, d, pl.ds(off, _L)] for d in range(_D)]
                        pid = sidv[t, pl.ds(off, _L)]
                        g1 = sgv[t, pl.ds(off, _L)]
                        i1 = lax.shift_right_logical(pid, 6)
                        i2 = jnp.bitwise_and(pid, 63)
                        hk1 = expert_step(i1 * _PT_ROW, x, list(h), g1)
                        hk2 = expert_step(i2 * _PT_ROW, x, list(h), 1.0 - g1)
                        return tuple(a + b for a, b in zip(hk1, hk2))

                    h = lax.fori_loop(0, _SEG, step, tuple(h), unroll=2)
                    for d in range(_D):
                        hv[d, pl.ds(off, _L)] = h[d]
                    return 0

                lax.fori_loop(0, _NGRP, grp_body, 0)
                return 0

            lax.fori_loop(0, _NSEG, seg_body, 0)

            def head_body(v, _):
                off = v * _L
                h = [hv[d, pl.ds(off, _L)] for d in range(_D)]
                hid = []
                for j in range(_HID):
                    a = hcv[128 + j] + h[0] * hcv[j]
                    for d in range(1, _D):
                        a = a + h[d] * hcv[d * _HID + j]
                    hid.append(_gelu_sc(a))
                for o in range(_FP2):
                    p = hcv[480 + o] + hid[0] * hcv[160 + o]
                    for j in range(1, _HID):
                        p = p + hid[j] * hcv[160 + j * _FP2 + o]
                    outv[o, pl.ds(off, _L)] = 1.0 / (1.0 + jnp.exp(-p))
                return 0

            lax.fori_loop(0, _NGRP, head_body, 0)
            pltpu.sync_copy(outv, out_h.at[wid, ch])
            return 0

        lax.fori_loop(0, _NCH, chunk_body, 0)

    return body(xr, sid, sg, ptab, hc)


def _liquid_tc_kernel(xT_ref, wrt_ref, p_ref, w1t_ref, b1_ref, w2t_ref,
                      b2_ref, out_ref):
    S = xT_ref.shape[0]
    Bt = xT_ref.shape[2]
    E = wrt_ref.shape[0]

    wrt = wrt_ref[...]
    P = p_ref[...]
    iota = jax.lax.broadcasted_iota(jnp.int32, (E, Bt), 0)
    big = jnp.int32(E)
    neg_inf = jnp.float32(-jnp.inf)

    def expert_apply(oh, x4, h8):
        pk = jnp.dot(P, oh, preferred_element_type=jnp.float32)
        pre = pk[64:72]
        for d in range(4):
            pre = pre + x4[d:d + 1] * pk[8 * d:8 * d + 8] \
                      + h8[d:d + 1] * pk[32 + 8 * d:40 + 8 * d]
        act = jnp.tanh(pre)
        return h8 + pk[72:80] * (act - h8)

    def step(t, h8):
        x4 = xT_ref[t]
        logits = wrt[:, 0:1] * x4[0:1]
        for d in range(1, 4):
            logits = logits + wrt[:, d:d + 1] * x4[d:d + 1]

        m1 = jnp.max(logits, axis=0, keepdims=True)
        idx1 = jnp.min(jnp.where(logits == m1, iota, big), axis=0,
                       keepdims=True)
        oh1 = (iota == idx1)
        masked = jnp.where(oh1, neg_inf, logits)
        m2 = jnp.max(masked, axis=0, keepdims=True)
        idx2 = jnp.min(jnp.where(masked == m2, iota, big), axis=0,
                       keepdims=True)
        oh2 = (iota == idx2)

        e2 = jnp.exp(m2 - m1)
        g1 = 1.0 / (1.0 + e2)
        g2 = e2 * g1

        hk1 = expert_apply(oh1.astype(jnp.float32), x4, h8)
        hk2 = expert_apply(oh2.astype(jnp.float32), x4, h8)
        return g1 * hk1 + g2 * hk2

    h0 = jnp.zeros((8, Bt), dtype=jnp.float32)
    h8 = jax.lax.fori_loop(0, S, step, h0)

    hidden = jnp.dot(w1t_ref[...], h8, preferred_element_type=jnp.float32)
    hidden = jax.nn.gelu(hidden + b1_ref[...])
    pred = jnp.dot(w2t_ref[...], hidden, preferred_element_type=jnp.float32)
    out_ref[...] = jax.nn.sigmoid(pred + b2_ref[...])


def _tc_path(x_tc, W_router, W_in, W_rec, b, log_tau, head_W1, head_b1,
             head_W2, head_b2):
    B, S, D = x_tc.shape
    E = W_router.shape[1]
    HID = head_W1.shape[1]
    FP2 = head_W2.shape[1]

    xT = jnp.transpose(x_tc, (1, 2, 0))

    def pack_dd(W):
        Wt = jnp.transpose(W, (1, 2, 0))
        Wt = jnp.pad(Wt, ((0, 0), (0, 4), (0, 0)))
        return Wt.reshape(8 * D, E)

    b8 = jnp.pad(b.T, ((0, 4), (0, 0)))
    dt8 = jnp.pad(_DT * jnp.exp(-log_tau).T, ((0, 4), (0, 0)))
    P = jnp.concatenate([pack_dd(W_in), pack_dd(W_rec), b8, dt8], axis=0)

    wrt = W_router.T
    w1t = jnp.pad(head_W1.T, ((0, 0), (0, 4)))
    b1c = head_b1.reshape(HID, 1)
    w2t = jnp.pad(head_W2.T, ((0, _OUTR - FP2), (0, 0)))
    b2c = jnp.pad(head_b2, (0, _OUTR - FP2)).reshape(_OUTR, 1)

    bt = min(_B_TILE, B)
    grid = (B // bt,)

    out = pl.pallas_call(
        _liquid_tc_kernel,
        grid=grid,
        in_specs=[
            pl.BlockSpec((S, D, bt), lambda i: (0, 0, i)),
            pl.BlockSpec((E, D), lambda i: (0, 0)),
            pl.BlockSpec((80, E), lambda i: (0, 0)),
            pl.BlockSpec((HID, 8), lambda i: (0, 0)),
            pl.BlockSpec((HID, 1), lambda i: (0, 0)),
            pl.BlockSpec((_OUTR, HID), lambda i: (0, 0)),
            pl.BlockSpec((_OUTR, 1), lambda i: (0, 0)),
        ],
        out_specs=pl.BlockSpec((_OUTR, bt), lambda i: (0, i)),
        out_shape=jax.ShapeDtypeStruct((_OUTR, B), jnp.float32),
        compiler_params=pltpu.CompilerParams(
            dimension_semantics=("parallel",)),
    )(xT, wrt, P, w1t, b1c, w2t, b2c)

    return out[:FP2].T.reshape(B, FP2 // 2, 2)


def kernel(x_seq, W_router, W_in, W_rec, b, log_tau, head_W1, head_b1,
           head_W2, head_b2):
    x_sc = x_seq[:_B_SC]
    x_tc = x_seq[_B_SC:]

    xr = x_sc.reshape(_NW, _NCH, _CT, _NSEG, _SEG, _D)
    xr = jnp.transpose(xr, (0, 1, 3, 4, 5, 2))

    sid, sg = _route(xr.reshape(_NW * _NCH * _NSEG, _SEG, _D, _CT),
                     W_router.T)
    sid = sid.reshape(_NW, _NCH, _NSEG, _SEG, _CT)
    sg = sg.reshape(_NW, _NCH, _NSEG, _SEG, _CT)

    ptab = jnp.concatenate(
        [W_in.reshape(_E, 16), W_rec.reshape(_E, 16), b,
         _DT * jnp.exp(-log_tau)], axis=1).reshape(-1)
    hc = jnp.concatenate(
        [head_W1.reshape(-1), head_b1, head_W2.reshape(-1), head_b2,
         jnp.zeros((512 - 490,), jnp.float32)])
    hc = jnp.tile(hc.reshape(512, 1), (1, _L))

    out_sc = _sc_recurrence(xr, sid, sg, ptab, hc)
    y_sc = jnp.transpose(out_sc, (0, 1, 3, 2)).reshape(_B_SC, _OUTR)
    y_sc = y_sc[:, :_FP2].reshape(_B_SC, _FP2 // 2, 2)

    y_tc = _tc_path(x_tc, W_router, W_in, W_rec, b, log_tau,
                    head_W1, head_b1, head_W2, head_b2)

    return jnp.concatenate([y_sc, y_tc], axis=0)


# speedup vs baseline: 1.7760x; 1.0004x over previous
import functools
import jax
import jax.numpy as jnp
from jax import lax
from jax.experimental import pallas as pl
from jax.experimental.pallas import tpu as pltpu
from jax.experimental.pallas import tpu_sc as plsc

_DT = 0.02

_NC, _NS, _L = 2, 16, 16
_NW = _NC * _NS
_B, _S, _D, _E = 16384, 200, 4, 64
_CT = 128
_B_SC = 8192
_NCH = _B_SC // (_NW * _CT)
_SEG = 40
_NSEG = _S // _SEG
_NGRP = _CT // _L
_PT_ROW = 40
_OUTR = 16
_HID, _FP2 = 32, 10
_B_TILE = 1024


def _route_kernel(xr_ref, wrt_ref, sid_ref, sg_ref):
    wrt = wrt_ref[...]
    iota = jax.lax.broadcasted_iota(jnp.int32, (_E, _CT), 0)
    big = jnp.int32(_E)
    neg_inf = jnp.float32(-jnp.inf)

    def body(s, _):
        x4 = xr_ref[0, s]
        logits = wrt[:, 0:1] * x4[0:1]
        for d in range(1, _D):
            logits = logits + wrt[:, d:d + 1] * x4[d:d + 1]
        m1 = jnp.max(logits, axis=0, keepdims=True)
        idx1 = jnp.min(jnp.where(logits == m1, iota, big), axis=0,
                       keepdims=True)
        masked = jnp.where(iota == idx1, neg_inf, logits)
        m2 = jnp.max(masked, axis=0, keepdims=True)
        idx2 = jnp.min(jnp.where(masked == m2, iota, big), axis=0,
                       keepdims=True)
        sid_ref[0, s] = ((idx1 << 6) | idx2)[0]
        sg_ref[0, s] = (1.0 / (1.0 + jnp.exp(m2 - m1)))[0]
        return 0

    lax.fori_loop(0, _SEG, body, 0, unroll=4)


def _route(xrf, wrt):
    nblk = xrf.shape[0]
    return pl.pallas_call(
        _route_kernel,
        grid=(nblk,),
        in_specs=[
            pl.BlockSpec((1, _SEG, _D, _CT), lambda i: (i, 0, 0, 0)),
            pl.BlockSpec((_E, _D), lambda i: (0, 0)),
        ],
        out_specs=[
            pl.BlockSpec((1, _SEG, _CT), lambda i: (i, 0, 0)),
            pl.BlockSpec((1, _SEG, _CT), lambda i: (i, 0, 0)),
        ],
        out_shape=[
            jax.ShapeDtypeStruct((nblk, _SEG, _CT), jnp.int32),
            jax.ShapeDtypeStruct((nblk, _SEG, _CT), jnp.float32),
        ],
        compiler_params=pltpu.CompilerParams(
            dimension_semantics=("parallel",)),
    )(xrf, wrt)


def _tanh_sc(x):
    return 1.0 - 2.0 / (jnp.exp(x + x) + 1.0)


def _gelu_sc(x):
    c0, c1 = 0.7978845608028654, 0.044715
    return 0.5 * x * (1.0 + _tanh_sc(c0 * (x + c1 * x * x * x)))


def _sc_recurrence(xr, sid, sg, ptab, hc):
    mesh = plsc.VectorSubcoreMesh(core_axis_name="c", subcore_axis_name="s")

    @functools.partial(
        pl.kernel, mesh=mesh,
        out_type=jax.ShapeDtypeStruct((_NW, _NCH, _OUTR, _CT), jnp.float32),
        scratch_types=[
            pltpu.VMEM((_SEG, _D, _CT), jnp.float32),
            pltpu.VMEM((_SEG, _CT), jnp.int32),
            pltpu.VMEM((_SEG, _CT), jnp.float32),
            pltpu.VMEM((_E * _PT_ROW,), jnp.float32),
            pltpu.VMEM((512, _L), jnp.float32),
            pltpu.VMEM((_D, _CT), jnp.float32),
            pltpu.VMEM((_OUTR, _CT), jnp.float32),
        ],
        compiler_params=pltpu.CompilerParams(needs_layout_passes=False),
    )
    def body(xr_h, sid_h, sg_h, ptab_h, hc_h, out_h,
             xv, sidv, sgv, ptv, hcv, hv, outv):
        wid = lax.axis_index("s") * _NC + lax.axis_index("c")
        pltpu.sync_copy(ptab_h, ptv)
        pltpu.sync_copy(hc_h, hcv)
        zero16 = jnp.zeros((_L,), jnp.float32)

        def expert_step(base, x, h, g):
            w = [plsc.load_gather(ptv, [base + j]) for j in range(_PT_ROW)]
            hk = []
            for e in range(_D):
                pre = w[32 + e]
                for d in range(_D):
                    pre = pre + x[d] * w[4 * d + e] + h[d] * w[16 + 4 * d + e]
                act = _tanh_sc(pre)
                hk.append(h[e] + w[36 + e] * (act - h[e]))
            return [g * v for v in hk]

        def chunk_body(ch, _):
            def zinit(v, _):
                off = v * _L
                for d in range(_D):
                    hv[d, pl.ds(off, _L)] = zero16
                return 0
            lax.fori_loop(0, _NGRP, zinit, 0)

            def seg_body(seg, _):
                pltpu.sync_copy(xr_h.at[wid, ch, seg], xv)
                pltpu.sync_copy(sid_h.at[wid, ch, seg], sidv)
                pltpu.sync_copy(sg_h.at[wid, ch, seg], sgv)

                def grp_body(v, _):
                    off = v * _L
                    h = [hv[d, pl.ds(off, _L)] for d in range(_D)]

                    def step(t, h):
                        x = [xv[t, d, pl.ds(off, _L)] for d in range(_D)]
                        pid = sidv[t, pl.ds(off, _L)]
                        g1 = sgv[t, pl.ds(off, _L)]
                        i1 = lax.shift_right_logical(pid, 6)
                        i2 = jnp.bitwise_and(pid, 63)
                        hk1 = expert_step(i1 * _PT_ROW, x, list(h), g1)
                        hk2 = expert_step(i2 * _PT_ROW, x, list(h), 1.0 - g1)
                        return tuple(a + b for a, b in zip(hk1, hk2))

                    h = lax.fori_loop(0, _SEG, step, tuple(h), unroll=4)
                    for d in range(_D):
                        hv[d, pl.ds(off, _L)] = h[d]
                    return 0

                lax.fori_loop(0, _NGRP, grp_body, 0)
                return 0

            lax.fori_loop(0, _NSEG, seg_body, 0)

            def head_body(v, _):
                off = v * _L
                h = [hv[d, pl.ds(off, _L)] for d in range(_D)]
                hid = []
                for j in range(_HID):
                    a = hcv[128 + j] + h[0] * hcv[j]
                    for d in range(1, _D):
                        a = a + h[d] * hcv[d * _HID + j]
                    hid.append(_gelu_sc(a))
                for o in range(_FP2):
                    p = hcv[480 + o] + hid[0] * hcv[160 + o]
                    for j in range(1, _HID):
                        p = p + hid[j] * hcv[160 + j * _FP2 + o]
                    outv[o, pl.ds(off, _L)] = 1.0 / (1.0 + jnp.exp(-p))
                return 0

            lax.fori_loop(0, _NGRP, head_body, 0)
            pltpu.sync_copy(outv, out_h.at[wid, ch])
            return 0

        lax.fori_loop(0, _NCH, chunk_body, 0)

    return body(xr, sid, sg, ptab, hc)


def _liquid_tc_kernel(xT_ref, wrt_ref, p_ref, w1t_ref, b1_ref, w2t_ref,
                      b2_ref, out_ref):
    S = xT_ref.shape[0]
    Bt = xT_ref.shape[2]
    E = wrt_ref.shape[0]

    wrt = wrt_ref[...]
    P = p_ref[...]
    iota = jax.lax.broadcasted_iota(jnp.int32, (E, Bt), 0)
    big = jnp.int32(E)
    neg_inf = jnp.float32(-jnp.inf)

    def expert_apply(oh, x4, h8):
        pk = jnp.dot(P, oh, preferred_element_type=jnp.float32)
        pre = pk[64:72]
        for d in range(4):
            pre = pre + x4[d:d + 1] * pk[8 * d:8 * d + 8] \
                      + h8[d:d + 1] * pk[32 + 8 * d:40 + 8 * d]
        act = jnp.tanh(pre)
        return h8 + pk[72:80] * (act - h8)

    def step(t, h8):
        x4 = xT_ref[t]
        logits = wrt[:, 0:1] * x4[0:1]
        for d in range(1, 4):
            logits = logits + wrt[:, d:d + 1] * x4[d:d + 1]

        m1 = jnp.max(logits, axis=0, keepdims=True)
        idx1 = jnp.min(jnp.where(logits == m1, iota, big), axis=0,
                       keepdims=True)
        oh1 = (iota == idx1)
        masked = jnp.where(oh1, neg_inf, logits)
        m2 = jnp.max(masked, axis=0, keepdims=True)
        idx2 = jnp.min(jnp.where(masked == m2, iota, big), axis=0,
                       keepdims=True)
        oh2 = (iota == idx2)

        e2 = jnp.exp(m2 - m1)
        g1 = 1.0 / (1.0 + e2)
        g2 = e2 * g1

        hk1 = expert_apply(oh1.astype(jnp.float32), x4, h8)
        hk2 = expert_apply(oh2.astype(jnp.float32), x4, h8)
        return g1 * hk1 + g2 * hk2

    h0 = jnp.zeros((8, Bt), dtype=jnp.float32)
    h8 = jax.lax.fori_loop(0, S, step, h0)

    hidden = jnp.dot(w1t_ref[...], h8, preferred_element_type=jnp.float32)
    hidden = jax.nn.gelu(hidden + b1_ref[...])
    pred = jnp.dot(w2t_ref[...], hidden, preferred_element_type=jnp.float32)
    out_ref[...] = jax.nn.sigmoid(pred + b2_ref[...])


def _tc_path(x_tc, W_router, W_in, W_rec, b, log_tau, head_W1, head_b1,
             head_W2, head_b2):
    B, S, D = x_tc.shape
    E = W_router.shape[1]
    HID = head_W1.shape[1]
    FP2 = head_W2.shape[1]

    xT = jnp.transpose(x_tc, (1, 2, 0))

    def pack_dd(W):
        Wt = jnp.transpose(W, (1, 2, 0))
        Wt = jnp.pad(Wt, ((0, 0), (0, 4), (0, 0)))
        return Wt.reshape(8 * D, E)

    b8 = jnp.pad(b.T, ((0, 4), (0, 0)))
    dt8 = jnp.pad(_DT * jnp.exp(-log_tau).T, ((0, 4), (0, 0)))
    P = jnp.concatenate([pack_dd(W_in), pack_dd(W_rec), b8, dt8], axis=0)

    wrt = W_router.T
    w1t = jnp.pad(head_W1.T, ((0, 0), (0, 4)))
    b1c = head_b1.reshape(HID, 1)
    w2t = jnp.pad(head_W2.T, ((0, _OUTR - FP2), (0, 0)))
    b2c = jnp.pad(head_b2, (0, _OUTR - FP2)).reshape(_OUTR, 1)

    bt = min(_B_TILE, B)
    grid = (B // bt,)

    out = pl.pallas_call(
        _liquid_tc_kernel,
        grid=grid,
        in_specs=[
            pl.BlockSpec((S, D, bt), lambda i: (0, 0, i)),
            pl.BlockSpec((E, D), lambda i: (0, 0)),
            pl.BlockSpec((80, E), lambda i: (0, 0)),
            pl.BlockSpec((HID, 8), lambda i: (0, 0)),
            pl.BlockSpec((HID, 1), lambda i: (0, 0)),
            pl.BlockSpec((_OUTR, HID), lambda i: (0, 0)),
            pl.BlockSpec((_OUTR, 1), lambda i: (0, 0)),
        ],
        out_specs=pl.BlockSpec((_OUTR, bt), lambda i: (0, i)),
        out_shape=jax.ShapeDtypeStruct((_OUTR, B), jnp.float32),
        compiler_params=pltpu.CompilerParams(
            dimension_semantics=("parallel",)),
    )(xT, wrt, P, w1t, b1c, w2t, b2c)

    return out[:FP2].T.reshape(B, FP2 // 2, 2)


def kernel(x_seq, W_router, W_in, W_rec, b, log_tau, head_W1, head_b1,
           head_W2, head_b2):
    x_sc = x_seq[:_B_SC]
    x_tc = x_seq[_B_SC:]

    xr = x_sc.reshape(_NW, _NCH, _CT, _NSEG, _SEG, _D)
    xr = jnp.transpose(xr, (0, 1, 3, 4, 5, 2))

    sid, sg = _route(xr.reshape(_NW * _NCH * _NSEG, _SEG, _D, _CT),
                     W_router.T)
    sid = sid.reshape(_NW, _NCH, _NSEG, _SEG, _CT)
    sg = sg.reshape(_NW, _NCH, _NSEG, _SEG, _CT)

    ptab = jnp.concatenate(
        [W_in.reshape(_E, 16), W_rec.reshape(_E, 16), b,
         _DT * jnp.exp(-log_tau)], axis=1).reshape(-1)
    hc = jnp.concatenate(
        [head_W1.reshape(-1), head_b1, head_W2.reshape(-1), head_b2,
         jnp.zeros((512 - 490,), jnp.float32)])
    hc = jnp.tile(hc.reshape(512, 1), (1, _L))

    out_sc = _sc_recurrence(xr, sid, sg, ptab, hc)
    y_sc = jnp.transpose(out_sc, (0, 1, 3, 2)).reshape(_B_SC, _OUTR)
    y_sc = y_sc[:, :_FP2].reshape(_B_SC, _FP2 // 2, 2)

    y_tc = _tc_path(x_tc, W_router, W_in, W_rec, b, log_tau,
                    head_W1, head_b1, head_W2, head_b2)

    return jnp.concatenate([y_sc, y_tc], axis=0)
